# Initial kernel scaffold; baseline (speedup 1.0000x reference)
import jax
import jax.numpy as jnp
from jax.experimental import pallas as pl


def kernel(x, encoder_output, params):
    raise NotImplementedError("write your pallas kernel here")


# speedup vs baseline: 2.5414x; 2.5414x over previous
import functools
import math

import jax
import jax.numpy as jnp
from jax.experimental import pallas as pl
from jax.experimental.pallas import tpu as pltpu

D = 768
H = 12
DH = D // H
FF = 2048
E = 8
S = 2048

_SCALE = 1.0 / math.sqrt(DH)
_INV_SQRT2 = 1.0 / math.sqrt(2.0)


def _qkv_body(xq_ref, xkv_ref, wq_ref, bq_ref, wk_ref, bk_ref, wv_ref, bv_ref,
              q_ref, k_ref, v_ref):
    xq = xq_ref[...]
    xkv = xkv_ref[...]
    q_ref[...] = jnp.dot(xq, wq_ref[...], preferred_element_type=jnp.float32) + bq_ref[...]
    k_ref[...] = jnp.dot(xkv, wk_ref[...], preferred_element_type=jnp.float32) + bk_ref[...]
    v_ref[...] = jnp.dot(xkv, wv_ref[...], preferred_element_type=jnp.float32) + bv_ref[...]


def _qkv(xq, xkv, wq, bq, wk, bk, wv, bv):
    shp = jax.ShapeDtypeStruct((S, D), jnp.float32)
    return pl.pallas_call(
        _qkv_body,
        out_shape=(shp, shp, shp),
    )(xq, xkv, wq, bq, wk, bk, wv, bv)


def _attn_body(q_ref, k_ref, v_ref, o_ref, *, causal, bq):
    qb = pl.program_id(0)
    if causal:
        row = qb * bq + jax.lax.broadcasted_iota(jnp.int32, (bq, S), 0)
        col = jax.lax.broadcasted_iota(jnp.int32, (bq, S), 1)
        keep = col <= row
    for h in range(H):
        sl = slice(h * DH, (h + 1) * DH)
        q = q_ref[:, sl]
        k = k_ref[:, sl]
        s = jax.lax.dot_general(q, k, (((1,), (1,)), ((), ())),
                                preferred_element_type=jnp.float32) * _SCALE
        if causal:
            s = jnp.where(keep, s, -jnp.inf)
        m = jnp.max(s, axis=-1, keepdims=True)
        p = jnp.exp(s - m)
        l = jnp.sum(p, axis=-1, keepdims=True)
        o_ref[:, sl] = jnp.dot(p / l, v_ref[:, sl],
                               preferred_element_type=jnp.float32)


def _attn(q, k, v, causal, bq=256):
    nq = S // bq
    return pl.pallas_call(
        functools.partial(_attn_body, causal=causal, bq=bq),
        grid=(nq,),
        in_specs=[
            pl.BlockSpec((bq, D), lambda t: (t, 0)),
            pl.BlockSpec((S, D), lambda t: (0, 0)),
            pl.BlockSpec((S, D), lambda t: (0, 0)),
        ],
        out_specs=pl.BlockSpec((bq, D), lambda t: (t, 0)),
        out_shape=jax.ShapeDtypeStruct((S, D), jnp.float32),
        compiler_params=pltpu.CompilerParams(
            dimension_semantics=("arbitrary",)),
    )(q, k, v)


def _proj_ln_body(a_ref, w_ref, b_ref, r_ref, g_ref, gb_ref, o_ref):
    y = jnp.dot(a_ref[...], w_ref[...], preferred_element_type=jnp.float32)
    y = y + b_ref[...] + r_ref[...]
    m = jnp.mean(y, axis=-1, keepdims=True)
    c = y - m
    var = jnp.mean(c * c, axis=-1, keepdims=True)
    o_ref[...] = c * jax.lax.rsqrt(var + 1e-5) * g_ref[...] + gb_ref[...]


def _proj_ln(a, w, b, resid, g, gb):
    return pl.pallas_call(
        _proj_ln_body,
        out_shape=jax.ShapeDtypeStruct((S, D), jnp.float32),
    )(a, w, b, resid, g, gb)


def _router_body(x_ref, w_ref, b_ref, g_ref):
    lg = jnp.dot(x_ref[...], w_ref[...], preferred_element_type=jnp.float32,
                 precision=jax.lax.Precision.HIGHEST)
    lg = lg + b_ref[...]
    lane = jax.lax.broadcasted_iota(jnp.int32, (S, E), 1)
    m1 = jnp.max(lg, axis=-1, keepdims=True)
    i1 = jnp.min(jnp.where(lg == m1, lane, E), axis=-1, keepdims=True)
    mask1 = lane == i1
    lgm = jnp.where(mask1, -jnp.inf, lg)
    m2 = jnp.max(lgm, axis=-1, keepdims=True)
    i2 = jnp.min(jnp.where(lgm == m2, lane, E), axis=-1, keepdims=True)
    mask2 = lane == i2
    ex = jnp.exp(lg - m1)
    z = jnp.sum(ex, axis=-1, keepdims=True)
    p = ex / z
    v1 = jnp.sum(jnp.where(mask1, p, 0.0), axis=-1, keepdims=True)
    v2 = jnp.sum(jnp.where(mask2, p, 0.0), axis=-1, keepdims=True)
    e2 = jnp.exp(v2 - v1)
    w1 = 1.0 / (1.0 + e2)
    w2 = e2 / (1.0 + e2)
    g_ref[...] = jnp.where(mask1, w1, 0.0) + jnp.where(mask2, w2, 0.0)


def _router(x, w, b):
    return pl.pallas_call(
        _router_body,
        out_shape=jax.ShapeDtypeStruct((S, E), jnp.float32),
    )(x, w, b)


def _moe_body(x_ref, g_ref, w1_ref, b1_ref, w2_ref, b2_ref, g3_ref, b3_ref,
              o_ref, *, bt, nt):
    e = pl.program_id(0)
    tb = pl.program_id(1)
    xb = x_ref[...]
    h = jnp.dot(xb, w1_ref[0], preferred_element_type=jnp.float32) + b1_ref[0]
    h = 0.5 * h * (1.0 + jax.lax.erf(h * _INV_SQRT2))
    y = jnp.dot(h, w2_ref[0], preferred_element_type=jnp.float32) + b2_ref[0]
    gb = g_ref[pl.ds(tb * bt, bt), :]
    lane = jax.lax.broadcasted_iota(jnp.int32, (bt, E), 1)
    ge = jnp.sum(jnp.where(lane == e, gb, 0.0), axis=-1, keepdims=True)
    contrib = ge * y
    sl = pl.ds(tb * bt, bt)

    @pl.when(e == 0)
    def _():
        o_ref[sl, :] = xb + contrib

    @pl.when(e != 0)
    def _():
        o_ref[sl, :] = o_ref[sl, :] + contrib

    @pl.when((e == E - 1) & (tb == nt - 1))
    def _():
        yf = o_ref[...]
        m = jnp.mean(yf, axis=-1, keepdims=True)
        c = yf - m
        var = jnp.mean(c * c, axis=-1, keepdims=True)
        o_ref[...] = c * jax.lax.rsqrt(var + 1e-5) * g3_ref[...] + b3_ref[...]


def _moe(x, gates, w1, b1, w2, b2, g3, b3, bt=512):
    nt = S // bt
    out = pl.pallas_call(
        functools.partial(_moe_body, bt=bt, nt=nt),
        grid=(E, nt),
        in_specs=[
            pl.BlockSpec((bt, D), lambda e, t: (t, 0)),
            pl.BlockSpec((S, E), lambda e, t: (0, 0)),
            pl.BlockSpec((1, D, FF), lambda e, t: (e, 0, 0)),
            pl.BlockSpec((1, 1, FF), lambda e, t: (e, 0, 0)),
            pl.BlockSpec((1, FF, D), lambda e, t: (e, 0, 0)),
            pl.BlockSpec((1, 1, D), lambda e, t: (e, 0, 0)),
            pl.BlockSpec((1, D), lambda e, t: (0, 0)),
            pl.BlockSpec((1, D), lambda e, t: (0, 0)),
        ],
        out_specs=pl.BlockSpec((S, D), lambda e, t: (0, 0)),
        out_shape=jax.ShapeDtypeStruct((S, D), jnp.float32),
        compiler_params=pltpu.CompilerParams(
            dimension_semantics=("arbitrary", "arbitrary")),
    )(x, gates, w1, b1.reshape(E, 1, FF), w2, b2.reshape(E, 1, D),
      g3.reshape(1, D), b3.reshape(1, D))
    return out


def kernel(x, encoder_output, params):
    p = params
    x0 = x[0]
    enc = encoder_output[0]

    q, k, v = _qkv(x0, x0, p['sa_q_w'], p['sa_q_b'], p['sa_k_w'], p['sa_k_b'],
                   p['sa_v_w'], p['sa_v_b'])
    sa = _attn(q, k, v, causal=True)
    x1 = _proj_ln(sa, p['sa_o_w'], p['sa_o_b'], x0, p['ln1_g'], p['ln1_b'])

    q, k, v = _qkv(x1, enc, p['ca_q_w'], p['ca_q_b'], p['ca_k_w'], p['ca_k_b'],
                   p['ca_v_w'], p['ca_v_b'])
    ca = _attn(q, k, v, causal=False)
    x2 = _proj_ln(ca, p['ca_o_w'], p['ca_o_b'], x1, p['ln2_g'], p['ln2_b'])

    gates = _router(x2, p['router_w'], p['router_b'])
    out = _moe(x2, gates, p['moe_w1'], p['moe_b1'], p['moe_w2'], p['moe_b2'],
               p['ln3_g'], p['ln3_b'])
    return out[None]


# speedup vs baseline: 2.6511x; 1.0431x over previous
import functools
import math

import jax
import jax.numpy as jnp
from jax import lax
from jax.experimental import pallas as pl
from jax.experimental.pallas import tpu as pltpu
from jax.experimental.pallas import tpu_sc as plsc

D = 768
H = 12
DH = D // H
FF = 2048
E = 8
S = 2048

BT = 256
NB = 24
NROWS = NB * BT
NW = 32
TPW = S // NW

_SCALE = 1.0 / math.sqrt(DH)
_INV_SQRT2 = 1.0 / math.sqrt(2.0)


def _qkv_body(xq_ref, xkv_ref, wq_ref, bq_ref, wk_ref, bk_ref, wv_ref, bv_ref,
              q_ref, k_ref, v_ref):
    xq = xq_ref[...]
    xkv = xkv_ref[...]
    q_ref[...] = jnp.dot(xq, wq_ref[...],
                         preferred_element_type=jnp.float32) + bq_ref[...]
    k_ref[...] = jnp.dot(xkv, wk_ref[...],
                         preferred_element_type=jnp.float32) + bk_ref[...]
    v_ref[...] = jnp.dot(xkv, wv_ref[...],
                         preferred_element_type=jnp.float32) + bv_ref[...]


def _qkv(xq, xkv, wq, bq, wk, bk, wv, bv):
    shp = jax.ShapeDtypeStruct((S, D), jnp.float32)
    return pl.pallas_call(
        _qkv_body,
        out_shape=(shp, shp, shp),
    )(xq, xkv, wq, bq, wk, bk, wv, bv)


def _attn_body(q_ref, k_ref, v_ref, o_ref, *, causal, bq):
    qb = pl.program_id(0)
    if causal:
        row = qb * bq + jax.lax.broadcasted_iota(jnp.int32, (bq, S), 0)
        col = jax.lax.broadcasted_iota(jnp.int32, (bq, S), 1)
        keep = col <= row
    for h in range(H):
        sl = slice(h * DH, (h + 1) * DH)
        q = q_ref[:, sl]
        k = k_ref[:, sl]
        s = jax.lax.dot_general(q, k, (((1,), (1,)), ((), ())),
                                preferred_element_type=jnp.float32) * _SCALE
        if causal:
            s = jnp.where(keep, s, -jnp.inf)
        m = jnp.max(s, axis=-1, keepdims=True)
        p = jnp.exp(s - m)
        l = jnp.sum(p, axis=-1, keepdims=True)
        o_ref[:, sl] = jnp.dot(p / l, v_ref[:, sl],
                               preferred_element_type=jnp.float32)


def _attn(q, k, v, causal, bq=256):
    nq = S // bq
    return pl.pallas_call(
        functools.partial(_attn_body, causal=causal, bq=bq),
        grid=(nq,),
        in_specs=[
            pl.BlockSpec((bq, D), lambda t: (t, 0)),
            pl.BlockSpec((S, D), lambda t: (0, 0)),
            pl.BlockSpec((S, D), lambda t: (0, 0)),
        ],
        out_specs=pl.BlockSpec((bq, D), lambda t: (t, 0)),
        out_shape=jax.ShapeDtypeStruct((S, D), jnp.float32),
        compiler_params=pltpu.CompilerParams(
            dimension_semantics=("arbitrary",)),
    )(q, k, v)


def _proj_ln_body(a_ref, w_ref, b_ref, r_ref, g_ref, gb_ref, o_ref):
    y = jnp.dot(a_ref[...], w_ref[...], preferred_element_type=jnp.float32)
    y = y + b_ref[...] + r_ref[...]
    m = jnp.mean(y, axis=-1, keepdims=True)
    c = y - m
    var = jnp.mean(c * c, axis=-1, keepdims=True)
    o_ref[...] = c * jax.lax.rsqrt(var + 1e-5) * g_ref[...] + gb_ref[...]


def _proj_ln(a, w, b, resid, g, gb):
    return pl.pallas_call(
        _proj_ln_body,
        out_shape=jax.ShapeDtypeStruct((S, D), jnp.float32),
    )(a, w, b, resid, g, gb)


def _router_body(x_ref, w_ref, b_ref,
                 d1_ref, d2_ref, w1_ref, w2_ref, be_ref, bv_ref):
    lg = jnp.dot(x_ref[...], w_ref[...], preferred_element_type=jnp.float32)
    lg = lg + b_ref[...]
    lane = jax.lax.broadcasted_iota(jnp.int32, (S, E), 1)
    m1 = jnp.max(lg, axis=-1, keepdims=True)
    i1 = jnp.min(jnp.where(lg == m1, lane, E), axis=-1, keepdims=True)
    mask1 = lane == i1
    lgm = jnp.where(mask1, -jnp.inf, lg)
    m2 = jnp.max(lgm, axis=-1, keepdims=True)
    i2 = jnp.min(jnp.where(lgm == m2, lane, E), axis=-1, keepdims=True)
    mask2 = lane == i2
    ex = jnp.exp(lg - m1)
    z = jnp.sum(ex, axis=-1, keepdims=True)
    p = ex / z
    v1 = jnp.sum(jnp.where(mask1, p, 0.0), axis=-1, keepdims=True)
    v2 = jnp.sum(jnp.where(mask2, p, 0.0), axis=-1, keepdims=True)
    e2 = jnp.exp(v2 - v1)
    w1_ref[...] = 1.0 / (1.0 + e2)
    w2_ref[...] = e2 / (1.0 + e2)

    occ = jnp.where(mask1 | mask2, 1.0, 0.0)
    r_i = jax.lax.broadcasted_iota(jnp.int32, (S, S), 0)
    c_i = jax.lax.broadcasted_iota(jnp.int32, (S, S), 1)
    tril = jnp.where(r_i > c_i, 1.0, 0.0)
    rank = jnp.dot(tril, occ, preferred_element_type=jnp.float32)
    counts = jnp.sum(occ, axis=0, keepdims=True)
    cpad = jnp.floor((counts + (BT - 1.0)) * (1.0 / BT)) * BT
    base = jnp.zeros((1, E), jnp.float32)
    for k in range(1, E):
        shifted = jnp.concatenate(
            [jnp.zeros((1, k), jnp.float32), cpad[:, :E - k]], axis=1)
        base = base + shifted
    dest = base + rank
    d1 = jnp.sum(jnp.where(mask1, dest, 0.0), axis=-1, keepdims=True)
    d2 = jnp.sum(jnp.where(mask2, dest, 0.0), axis=-1, keepdims=True)
    d1_ref[...] = d1.astype(jnp.int32)
    d2_ref[...] = d2.astype(jnp.int32)

    bi = jax.lax.broadcasted_iota(jnp.int32, (NB, E), 0).astype(jnp.float32) * BT
    ei = jax.lax.broadcasted_iota(jnp.int32, (NB, E), 1)
    inb = (bi >= base) & (bi < base + cpad)
    blk_e = jnp.sum(jnp.where(inb, ei, 0), axis=-1, keepdims=True)
    blk_v = jnp.sum(jnp.where(inb, 1, 0), axis=-1, keepdims=True)
    lastexp = jnp.max(jnp.where(inb, ei, 0))
    be_ref[...] = jnp.where(blk_v > 0, blk_e, lastexp).astype(jnp.int32)
    bv_ref[...] = blk_v.astype(jnp.int32)


def _router(x, w, b):
    f32 = jnp.float32
    i32 = jnp.int32
    return pl.pallas_call(
        _router_body,
        out_shape=(
            jax.ShapeDtypeStruct((S, 1), i32),
            jax.ShapeDtypeStruct((S, 1), i32),
            jax.ShapeDtypeStruct((S, 1), f32),
            jax.ShapeDtypeStruct((S, 1), f32),
            jax.ShapeDtypeStruct((NB, 1), i32),
            jax.ShapeDtypeStruct((NB, 1), i32),
        ),
    )(x, w, b)


def _sc_mesh():
    return plsc.VectorSubcoreMesh(core_axis_name="c", subcore_axis_name="s",
                                  num_cores=2, num_subcores=16)


def _sc_dispatch_body(x2_hbm, d1_hbm, d2_hbm, xdisp_hbm, dvec, rows, sem):
    wid = lax.axis_index("s") * 2 + lax.axis_index("c")
    base = wid * TPW
    pltpu.sync_copy(x2_hbm.at[pl.ds(base, TPW)], rows)
    pltpu.sync_copy(d1_hbm.at[wid], dvec)
    pltpu.async_copy(rows, xdisp_hbm.at[dvec], sem).wait()
    pltpu.sync_copy(d2_hbm.at[wid], dvec)
    pltpu.async_copy(rows, xdisp_hbm.at[dvec], sem).wait()


def _sc_dispatch(x2, d1r, d2r):
    fn = pl.kernel(
        _sc_dispatch_body,
        out_type=jax.ShapeDtypeStruct((NROWS, D), jnp.float32),
        mesh=_sc_mesh(),
        scratch_types=[
            pltpu.VMEM((TPW,), jnp.int32),
            pltpu.VMEM((TPW, D), jnp.float32),
            pltpu.SemaphoreType.DMA,
        ],
    )
    return fn(x2, d1r, d2r)


def _sc_combine_body(y_hbm, d1_hbm, d2_hbm, y1_hbm, y2_hbm, dvec, rows, sem):
    wid = lax.axis_index("s") * 2 + lax.axis_index("c")
    base = wid * TPW
    pltpu.sync_copy(d1_hbm.at[wid], dvec)
    pltpu.async_copy(y_hbm.at[dvec], rows, sem).wait()
    pltpu.sync_copy(rows, y1_hbm.at[pl.ds(base, TPW)])
    pltpu.sync_copy(d2_hbm.at[wid], dvec)
    pltpu.async_copy(y_hbm.at[dvec], rows, sem).wait()
    pltpu.sync_copy(rows, y2_hbm.at[pl.ds(base, TPW)])


def _sc_combine(y, d1r, d2r):
    fn = pl.kernel(
        _sc_combine_body,
        out_type=(jax.ShapeDtypeStruct((S, D), jnp.float32),
                  jax.ShapeDtypeStruct((S, D), jnp.float32)),
        mesh=_sc_mesh(),
        scratch_types=[
            pltpu.VMEM((TPW,), jnp.int32),
            pltpu.VMEM((TPW, D), jnp.float32),
            pltpu.SemaphoreType.DMA,
        ],
    )
    return fn(y, d1r, d2r)


def _gmm_body(be_ref, bv_ref, x_ref, w1_ref, b1_ref, w2_ref, b2_ref, o_ref):
    i = pl.program_id(0)

    @pl.when(bv_ref[i] != 0)
    def _():
        h = jnp.dot(x_ref[...], w1_ref[0],
                    preferred_element_type=jnp.float32) + b1_ref[0]
        h = 0.5 * h * (1.0 + jax.lax.erf(h * _INV_SQRT2))
        o_ref[...] = jnp.dot(h, w2_ref[0],
                             preferred_element_type=jnp.float32) + b2_ref[0]


def _gmm(xdisp, blk_e, blk_v, w1, b1, w2, b2):
    grid_spec = pltpu.PrefetchScalarGridSpec(
        num_scalar_prefetch=2,
        grid=(NB,),
        in_specs=[
            pl.BlockSpec((BT, D), lambda i, be, bv: (i, 0)),
            pl.BlockSpec((1, D, FF), lambda i, be, bv: (be[i], 0, 0)),
            pl.BlockSpec((1, 1, FF), lambda i, be, bv: (be[i], 0, 0)),
            pl.BlockSpec((1, FF, D), lambda i, be, bv: (be[i], 0, 0)),
            pl.BlockSpec((1, 1, D), lambda i, be, bv: (be[i], 0, 0)),
        ],
        out_specs=pl.BlockSpec((BT, D), lambda i, be, bv: (i, 0)),
    )
    return pl.pallas_call(
        _gmm_body,
        grid_spec=grid_spec,
        out_shape=jax.ShapeDtypeStruct((NROWS, D), jnp.float32),
        compiler_params=pltpu.CompilerParams(
            dimension_semantics=("arbitrary",)),
    )(blk_e, blk_v, xdisp, w1, b1.reshape(E, 1, FF), w2, b2.reshape(E, 1, D))


def _combine_ln_body(x_ref, y1_ref, y2_ref, w1_ref, w2_ref, g_ref, b_ref,
                     o_ref):
    yy = x_ref[...] + w1_ref[...] * y1_ref[...] + w2_ref[...] * y2_ref[...]
    m = jnp.mean(yy, axis=-1, keepdims=True)
    c = yy - m
    var = jnp.mean(c * c, axis=-1, keepdims=True)
    o_ref[...] = c * jax.lax.rsqrt(var + 1e-5) * g_ref[...] + b_ref[...]


def _combine_ln(x2, y1, y2, w1, w2, g3, b3):
    return pl.pallas_call(
        _combine_ln_body,
        out_shape=jax.ShapeDtypeStruct((S, D), jnp.float32),
    )(x2, y1, y2, w1, w2, g3.reshape(1, D), b3.reshape(1, D))


def kernel(x, encoder_output, params):
    p = params
    x0 = x[0]
    enc = encoder_output[0]

    q, k, v = _qkv(x0, x0, p['sa_q_w'], p['sa_q_b'], p['sa_k_w'], p['sa_k_b'],
                   p['sa_v_w'], p['sa_v_b'])
    sa = _attn(q, k, v, causal=True)
    x1 = _proj_ln(sa, p['sa_o_w'], p['sa_o_b'], x0, p['ln1_g'], p['ln1_b'])

    q, k, v = _qkv(x1, enc, p['ca_q_w'], p['ca_q_b'], p['ca_k_w'], p['ca_k_b'],
                   p['ca_v_w'], p['ca_v_b'])
    ca = _attn(q, k, v, causal=False)
    x2 = _proj_ln(ca, p['ca_o_w'], p['ca_o_b'], x1, p['ln2_g'], p['ln2_b'])

    d1, d2, wt1, wt2, be, bv = _router(x2, p['router_w'], p['router_b'])
    d1r = d1.reshape(NW, TPW)
    d2r = d2.reshape(NW, TPW)
    xdisp = _sc_dispatch(x2, d1r, d2r)
    ydisp = _gmm(xdisp, be.reshape(NB), bv.reshape(NB),
                 p['moe_w1'], p['moe_b1'], p['moe_w2'], p['moe_b2'])
    y1, y2 = _sc_combine(ydisp, d1r, d2r)
    out = _combine_ln(x2, y1, y2, wt1, wt2, p['ln3_g'], p['ln3_b'])
    return out[None]
